# Initial kernel scaffold; baseline (speedup 1.0000x reference)
import jax
import jax.numpy as jnp
from jax.experimental import pallas as pl


def kernel(x, edge_index, W1, b1, g1, be1, W2, b2, g2, be2, Wh, bh):
    raise NotImplementedError("write your pallas kernel here")


# speedup vs baseline: 9.7997x; 9.7997x over previous
import functools

import jax
import jax.numpy as jnp
from jax import lax
from jax.experimental import pallas as pl
from jax.experimental.pallas import tpu as pltpu
from jax.experimental.pallas import tpu_sc as plsc

N = 10000
NPAD = 10240
E = 160000
EPAD = 161792
CH = 128
PER_TILE = EPAD // 16
NCHUNK = PER_TILE // CH
DEAD = N
IN_DIM = 256
HID = 256
HALF = 128
ODIM = 128
EPS = 1e-5
ROWS_PER_TILE = NPAD // 16
ACC_ROWS = 10112
ACC_PER_TILE = ACC_ROWS // 16

f32 = jnp.float32

_mesh = plsc.VectorSubcoreMesh(core_axis_name="c", subcore_axis_name="s")


@functools.partial(
    pl.kernel,
    mesh=_mesh,
    out_type=(jax.ShapeDtypeStruct((NPAD, 16), f32),
              jax.ShapeDtypeStruct((NPAD, 16), f32)),
    scratch_types=[
        pltpu.VMEM((CH,), jnp.int32),
        pltpu.VMEM((CH, 16), f32),
        pltpu.VMEM((CH, 16), f32),
        pltpu.VMEM_SHARED((NPAD, 16), f32),
    ],
)
def _deg_kernel(src_hbm, dst_hbm, d0_hbm, d1_hbm, idx_v, ones_v, zeros_v, acc):
    cid = lax.axis_index("c")
    sid = lax.axis_index("s")

    def fill(i, carry):
        ones_v[i, :] = jnp.full((16,), 1.0, f32)
        zeros_v[i, :] = jnp.zeros((16,), f32)
        return carry

    lax.fori_loop(0, CH, fill, 0)

    def zinit(i, carry):
        pltpu.sync_copy(zeros_v, acc.at[pl.ds(sid * ROWS_PER_TILE + i * CH, CH)])
        return carry

    lax.fori_loop(0, ROWS_PER_TILE // CH, zinit, 0)
    plsc.subcore_barrier()

    def chunk(j, carry):
        base = pl.multiple_of(sid * PER_TILE + j * CH, 8)

        @pl.when(cid == 0)
        def _():
            pltpu.sync_copy(src_hbm.at[pl.ds(base, CH)], idx_v)

        @pl.when(cid == 1)
        def _():
            pltpu.sync_copy(dst_hbm.at[pl.ds(base, CH)], idx_v)

        pltpu.sync_copy(ones_v, acc.at[idx_v], add=True)
        return carry

    lax.fori_loop(0, NCHUNK, chunk, 0)
    plsc.subcore_barrier()

    out_slice = pl.ds(sid * ROWS_PER_TILE, ROWS_PER_TILE)

    @pl.when(cid == 0)
    def _():
        pltpu.sync_copy(acc.at[out_slice], d0_hbm.at[out_slice])

    @pl.when(cid == 1)
    def _():
        pltpu.sync_copy(acc.at[out_slice], d1_hbm.at[out_slice])


@functools.partial(
    pl.kernel,
    mesh=_mesh,
    out_type=(jax.ShapeDtypeStruct((NPAD, HALF), f32),
              jax.ShapeDtypeStruct((NPAD, HALF), f32)),
    scratch_types=[
        pltpu.VMEM((CH,), jnp.int32),
        pltpu.VMEM((CH,), jnp.int32),
        pltpu.VMEM((CH, HALF), f32),
        pltpu.VMEM((CH, HALF), f32),
        pltpu.VMEM((CH, HALF), f32),
        pltpu.VMEM_SHARED((ACC_ROWS, HALF), f32),
        pltpu.SemaphoreType.DMA,
        pltpu.SemaphoreType.DMA,
    ],
)
def _prop_kernel(y0_hbm, y1_hbm, src_hbm, dst_hbm, s0_hbm, s1_hbm,
                 sidx, didx, srow, drow, zrow, acc, sem_a, sem_b):
    cid = lax.axis_index("c")
    sid = lax.axis_index("s")

    def zfill(i, carry):
        def zcol(k, c2):
            zrow[i, pl.ds(k * 16, 16)] = jnp.zeros((16,), f32)
            return c2

        lax.fori_loop(0, HALF // 16, zcol, 0)
        return carry

    lax.fori_loop(0, CH, zfill, 0)

    def zinit(i, carry):
        pltpu.sync_copy(zrow, acc.at[pl.ds(sid * ACC_PER_TILE + i * CH, CH)])
        return carry

    lax.fori_loop(0, ACC_PER_TILE // CH, zinit, 0)
    rem = ACC_PER_TILE % CH
    pltpu.sync_copy(
        zrow.at[pl.ds(0, rem)],
        acc.at[pl.ds(sid * ACC_PER_TILE + (ACC_PER_TILE - rem), rem)])
    plsc.subcore_barrier()

    def chunk(j, carry):
        base = pl.multiple_of(sid * PER_TILE + j * CH, 8)
        pltpu.sync_copy(src_hbm.at[pl.ds(base, CH)], sidx)
        pltpu.sync_copy(dst_hbm.at[pl.ds(base, CH)], didx)

        @pl.when(cid == 0)
        def _():
            a = pltpu.async_copy(y0_hbm.at[sidx], srow, sem_a)
            b = pltpu.async_copy(y0_hbm.at[didx], drow, sem_b)
            a.wait()
            b.wait()

        @pl.when(cid == 1)
        def _():
            a = pltpu.async_copy(y1_hbm.at[sidx], srow, sem_a)
            b = pltpu.async_copy(y1_hbm.at[didx], drow, sem_b)
            a.wait()
            b.wait()

        pltpu.sync_copy(srow, acc.at[didx], add=True)
        pltpu.sync_copy(drow, acc.at[sidx], add=True)
        return carry

    lax.fori_loop(0, NCHUNK, chunk, 0)
    plsc.subcore_barrier()

    out_slice = pl.ds(sid * ACC_PER_TILE, ACC_PER_TILE)

    @pl.when(cid == 0)
    def _():
        pltpu.sync_copy(acc.at[out_slice], s0_hbm.at[out_slice])

    @pl.when(cid == 1)
    def _():
        pltpu.sync_copy(acc.at[out_slice], s1_hbm.at[out_slice])


TC_BLK = 512
HEAD_BLK = 400


def _dinv_of(d0_ref, d1_ref):
    deg = d0_ref[:, 0:1] + d1_ref[:, 0:1] + 1.0
    return lax.rsqrt(deg)


def _row_mask(nrows):
    i = pl.program_id(0)
    rows = i * nrows + lax.broadcasted_iota(jnp.int32, (nrows, 1), 0)
    return rows < N


def _tc1_body(x_ref, w_ref, b_ref, d0_ref, d1_ref, y0_ref, y1_ref):
    h = lax.dot_general(x_ref[...], w_ref[...], (((1,), (1,)), ((), ())),
                        preferred_element_type=f32)
    h = h + b_ref[...]
    y = h * _dinv_of(d0_ref, d1_ref)
    y = jnp.where(_row_mask(TC_BLK), y, 0.0)
    y0_ref[...] = y[:, :HALF]
    y1_ref[...] = y[:, HALF:]


def _tc1(x_p, W1, b1, d0, d1):
    return pl.pallas_call(
        _tc1_body,
        grid=(NPAD // TC_BLK,),
        in_specs=[
            pl.BlockSpec((TC_BLK, IN_DIM), lambda i: (i, 0)),
            pl.BlockSpec((HID, IN_DIM), lambda i: (0, 0)),
            pl.BlockSpec((1, HID), lambda i: (0, 0)),
            pl.BlockSpec((TC_BLK, 16), lambda i: (i, 0)),
            pl.BlockSpec((TC_BLK, 16), lambda i: (i, 0)),
        ],
        out_specs=[
            pl.BlockSpec((TC_BLK, HALF), lambda i: (i, 0)),
            pl.BlockSpec((TC_BLK, HALF), lambda i: (i, 0)),
        ],
        out_shape=[jax.ShapeDtypeStruct((NPAD, HALF), f32)] * 2,
    )(x_p, W1, b1, d0, d1)


def _stats_body(s0_ref, s1_ref, y0_ref, y1_ref, d0_ref, d1_ref, t_ref, st_ref):
    dinv = _dinv_of(d0_ref, d1_ref)
    t0 = (s0_ref[...] + y0_ref[...]) * dinv
    t1 = (s1_ref[...] + y1_ref[...]) * dinv
    t = jnp.concatenate([t0, t1], axis=1)
    t = jnp.where(_row_mask(TC_BLK), t, 0.0)
    t_ref[...] = t

    @pl.when(pl.program_id(0) == 0)
    def _():
        st_ref[...] = jnp.zeros_like(st_ref)

    upd = jnp.concatenate(
        [jnp.sum(t, axis=0, keepdims=True),
         jnp.sum(t * t, axis=0, keepdims=True),
         jnp.zeros((6, HID), f32)], axis=0)
    st_ref[...] = st_ref[...] + upd


def _tc_stats(s0, s1, y0, y1, d0, d1):
    return pl.pallas_call(
        _stats_body,
        grid=(NPAD // TC_BLK,),
        in_specs=[
            pl.BlockSpec((TC_BLK, HALF), lambda i: (i, 0)),
            pl.BlockSpec((TC_BLK, HALF), lambda i: (i, 0)),
            pl.BlockSpec((TC_BLK, HALF), lambda i: (i, 0)),
            pl.BlockSpec((TC_BLK, HALF), lambda i: (i, 0)),
            pl.BlockSpec((TC_BLK, 16), lambda i: (i, 0)),
            pl.BlockSpec((TC_BLK, 16), lambda i: (i, 0)),
        ],
        out_specs=[
            pl.BlockSpec((TC_BLK, HID), lambda i: (i, 0)),
            pl.BlockSpec((8, HID), lambda i: (0, 0)),
        ],
        out_shape=[jax.ShapeDtypeStruct((NPAD, HID), f32),
                   jax.ShapeDtypeStruct((8, HID), f32)],
    )(s0, s1, y0, y1, d0, d1)


def _bn_relu(t_ref, st_ref, g_ref, be_ref):
    mu = st_ref[0:1, :] * (1.0 / N)
    ex2 = st_ref[1:2, :] * (1.0 / N)
    rstd = lax.rsqrt(ex2 - mu * mu + EPS)
    xn = (t_ref[...] - mu) * rstd
    return jnp.maximum(xn * g_ref[...] + be_ref[...], 0.0)


def _mid_body(t_ref, st_ref, g_ref, be_ref, w_ref, b_ref, d0_ref, d1_ref,
              y0_ref, y1_ref):
    r = _bn_relu(t_ref, st_ref, g_ref, be_ref)
    h = lax.dot_general(r, w_ref[...], (((1,), (1,)), ((), ())),
                        preferred_element_type=f32)
    h = h + b_ref[...]
    y = h * _dinv_of(d0_ref, d1_ref)
    y = jnp.where(_row_mask(TC_BLK), y, 0.0)
    y0_ref[...] = y[:, :HALF]
    y1_ref[...] = y[:, HALF:]


def _tc_mid(t, st, g, be, W2, b2, d0, d1):
    return pl.pallas_call(
        _mid_body,
        grid=(NPAD // TC_BLK,),
        in_specs=[
            pl.BlockSpec((TC_BLK, HID), lambda i: (i, 0)),
            pl.BlockSpec((8, HID), lambda i: (0, 0)),
            pl.BlockSpec((1, HID), lambda i: (0, 0)),
            pl.BlockSpec((1, HID), lambda i: (0, 0)),
            pl.BlockSpec((HID, HID), lambda i: (0, 0)),
            pl.BlockSpec((1, HID), lambda i: (0, 0)),
            pl.BlockSpec((TC_BLK, 16), lambda i: (i, 0)),
            pl.BlockSpec((TC_BLK, 16), lambda i: (i, 0)),
        ],
        out_specs=[
            pl.BlockSpec((TC_BLK, HALF), lambda i: (i, 0)),
            pl.BlockSpec((TC_BLK, HALF), lambda i: (i, 0)),
        ],
        out_shape=[jax.ShapeDtypeStruct((NPAD, HALF), f32)] * 2,
    )(t, st, g, be, W2, b2, d0, d1)


def _head_body(t_ref, st_ref, g_ref, be_ref, w_ref, b_ref, o_ref):
    r = _bn_relu(t_ref, st_ref, g_ref, be_ref)
    o_ref[...] = lax.dot_general(r, w_ref[...], (((1,), (1,)), ((), ())),
                                 preferred_element_type=f32) + b_ref[...]


def _tc_head(t, st, g, be, Wh, bh):
    return pl.pallas_call(
        _head_body,
        grid=(N // HEAD_BLK,),
        in_specs=[
            pl.BlockSpec((HEAD_BLK, HID), lambda i: (i, 0)),
            pl.BlockSpec((8, HID), lambda i: (0, 0)),
            pl.BlockSpec((1, HID), lambda i: (0, 0)),
            pl.BlockSpec((1, HID), lambda i: (0, 0)),
            pl.BlockSpec((ODIM, HID), lambda i: (0, 0)),
            pl.BlockSpec((1, ODIM), lambda i: (0, 0)),
        ],
        out_specs=pl.BlockSpec((HEAD_BLK, ODIM), lambda i: (i, 0)),
        out_shape=jax.ShapeDtypeStruct((N, ODIM), f32),
    )(t, st, g, be, Wh, bh)


def kernel(x, edge_index, W1, b1, g1, be1, W2, b2, g2, be2, Wh, bh):
    pad = jnp.full((EPAD - E,), DEAD, jnp.int32)
    src_p = jnp.concatenate([edge_index[0], pad])
    dst_p = jnp.concatenate([edge_index[1], pad])
    x_p = jnp.concatenate([x, jnp.zeros((NPAD - N, IN_DIM), f32)])

    b1r = b1.reshape(1, HID)
    b2r = b2.reshape(1, HID)
    bhr = bh.reshape(1, ODIM)
    g1r = g1.reshape(1, HID)
    be1r = be1.reshape(1, HID)
    g2r = g2.reshape(1, HID)
    be2r = be2.reshape(1, HID)

    d0, d1 = _deg_kernel(src_p, dst_p)
    y0, y1 = _tc1(x_p, W1, b1r, d0, d1)
    s0, s1 = _prop_kernel(y0, y1, src_p, dst_p)
    t, st = _tc_stats(s0, s1, y0, y1, d0, d1)
    y0, y1 = _tc_mid(t, st, g1r, be1r, W2, b2r, d0, d1)
    s0, s1 = _prop_kernel(y0, y1, src_p, dst_p)
    t, st = _tc_stats(s0, s1, y0, y1, d0, d1)
    return _tc_head(t, st, g2r, be2r, Wh, bhr)
